# baseline (speedup 1.0000x reference)
import functools

import jax
import jax.numpy as jnp
from jax import lax
from jax.experimental import pallas as pl
from jax.experimental.pallas import tpu as pltpu
from jax.experimental.pallas import tpu_sc as plsc

N_NODES = 50000
D_FEAT = 256
EMBED = 128
BATCH = 16384
NSAMP = 32

NUM_WORKERS = 32
BPW = BATCH // NUM_WORKERS
GRP = 128
NGRP = BPW // GRP


def _proj_body(f_ref, w1_ref, w2_ref, p1_ref, p2_ref):
    f = f_ref[...].astype(jnp.bfloat16)
    p1_ref[...] = jnp.dot(f, w1_ref[...], preferred_element_type=jnp.float32)
    p2_ref[...] = jnp.dot(f, w2_ref[...], preferred_element_type=jnp.float32)


def _project(features, w1t, w2t):
    rows = 1000
    return pl.pallas_call(
        _proj_body,
        grid=(N_NODES // rows,),
        in_specs=[
            pl.BlockSpec((rows, D_FEAT), lambda i: (i, 0)),
            pl.BlockSpec((D_FEAT, EMBED), lambda i: (0, 0)),
            pl.BlockSpec((D_FEAT, EMBED), lambda i: (0, 0)),
        ],
        out_specs=[
            pl.BlockSpec((rows, EMBED), lambda i: (i, 0)),
            pl.BlockSpec((rows, EMBED), lambda i: (i, 0)),
        ],
        out_shape=[jax.ShapeDtypeStruct((N_NODES, EMBED), jnp.float32)] * 2,
    )(features, w1t, w2t)


def _relu_t_body(x_ref, o_ref):
    o_ref[...] = jnp.maximum(x_ref[...].T, 0.0)


def _relu_transpose(x):
    cols = 2048
    return pl.pallas_call(
        _relu_t_body,
        grid=(BATCH // cols,),
        in_specs=[pl.BlockSpec((cols, EMBED), lambda i: (i, 0))],
        out_specs=pl.BlockSpec((EMBED, cols), lambda i: (0, i)),
        out_shape=jax.ShapeDtypeStruct((EMBED, BATCH), jnp.float32),
    )(x)


def _sc_body(p1_hbm, p2_hbm, nodes_hbm, neight_hbm, out_hbm,
             nd_v, idx_v, acc_v, sem):
    wid = lax.axis_index("s") * 2 + lax.axis_index("c")
    base = wid * BPW

    nd_cp = pltpu.async_copy(nodes_hbm.at[pl.ds(base, BPW)], nd_v, sem)
    idx_cp = pltpu.async_copy(neight_hbm.at[:, pl.ds(base, BPW)], idx_v, sem)

    zero = jnp.zeros((16,), jnp.float32)

    def zero_rows(r, carry):
        for u in range(8):
            for f in range(EMBED // 16):
                acc_v[r * 8 + u, pl.ds(f * 16, 16)] = zero
        return carry

    lax.fori_loop(0, BPW // 8, zero_rows, 0)
    nd_cp.wait()
    idx_cp.wait()

    for q in range(NGRP):
        pltpu.async_copy(
            p1_hbm.at[nd_v.at[pl.ds(q * GRP, GRP)]],
            acc_v.at[pl.ds(q * GRP, GRP)], sem, add=True)

    def add_round(j, carry):
        for q in range(NGRP):
            pltpu.async_copy(
                p2_hbm.at[idx_v.at[j, pl.ds(q * GRP, GRP)]],
                acc_v.at[pl.ds(q * GRP, GRP)], sem, add=True)
        return carry

    lax.fori_loop(0, NSAMP, add_round, 0)

    def drain_round(j, carry):
        for q in range(NGRP):
            pltpu.make_async_copy(
                p2_hbm.at[idx_v.at[0, pl.ds(q * GRP, GRP)]],
                acc_v.at[pl.ds(q * GRP, GRP)], sem).wait()
        return carry

    lax.fori_loop(0, NSAMP + 1, drain_round, 0)

    pltpu.sync_copy(acc_v, out_hbm.at[pl.ds(base, BPW)])


_sc_gather = functools.partial(
    pl.kernel,
    mesh=plsc.VectorSubcoreMesh(core_axis_name="c", subcore_axis_name="s"),
    out_type=jax.ShapeDtypeStruct((BATCH, EMBED), jnp.float32),
    scratch_types=[
        pltpu.VMEM((BPW,), jnp.int32),
        pltpu.VMEM((NSAMP, BPW), jnp.int32),
        pltpu.VMEM((BPW, EMBED), jnp.float32),
        pltpu.SemaphoreType.DMA,
    ],
)(_sc_body)


def kernel(nodes, neigh_idx, features, weight):
    w1t = weight[:, :D_FEAT].T.astype(jnp.bfloat16)
    w2t = (weight[:, D_FEAT:].T * (1.0 / NSAMP)).astype(jnp.bfloat16)
    p1, p2 = _project(features, w1t, w2t)
    neight = neigh_idx.T.astype(jnp.int32)
    pre = _sc_gather(p1, p2, nodes.astype(jnp.int32), neight)
    return _relu_transpose(pre)


# speedup vs baseline: 1.0013x; 1.0013x over previous
import functools

import jax
import jax.numpy as jnp
from jax import lax
from jax.experimental import pallas as pl
from jax.experimental.pallas import tpu as pltpu
from jax.experimental.pallas import tpu_sc as plsc

N_NODES = 50000
D_FEAT = 256
EMBED = 128
BATCH = 16384
NSAMP = 32

NUM_WORKERS = 32
BPW = BATCH // NUM_WORKERS
GRP = 128
NGRP = BPW // GRP


def _proj_body(f_ref, w1_ref, w2_ref, p1_ref, p2_ref):
    f = f_ref[...]
    p1_ref[...] = jnp.dot(f, w1_ref[...], preferred_element_type=jnp.float32)
    p2_ref[...] = jnp.dot(f, w2_ref[...], preferred_element_type=jnp.float32)


def _project(features, w1t, w2t):
    rows = 1000
    return pl.pallas_call(
        _proj_body,
        grid=(N_NODES // rows,),
        in_specs=[
            pl.BlockSpec((rows, D_FEAT), lambda i: (i, 0)),
            pl.BlockSpec((D_FEAT, EMBED), lambda i: (0, 0)),
            pl.BlockSpec((D_FEAT, EMBED), lambda i: (0, 0)),
        ],
        out_specs=[
            pl.BlockSpec((rows, EMBED), lambda i: (i, 0)),
            pl.BlockSpec((rows, EMBED), lambda i: (i, 0)),
        ],
        out_shape=[jax.ShapeDtypeStruct((N_NODES, EMBED), jnp.float32)] * 2,
    )(features, w1t, w2t)


def _relu_t_body(x_ref, o_ref):
    o_ref[...] = jnp.maximum(x_ref[...].T, 0.0)


def _relu_transpose(x):
    cols = 2048
    return pl.pallas_call(
        _relu_t_body,
        grid=(BATCH // cols,),
        in_specs=[pl.BlockSpec((cols, EMBED), lambda i: (i, 0))],
        out_specs=pl.BlockSpec((EMBED, cols), lambda i: (0, i)),
        out_shape=jax.ShapeDtypeStruct((EMBED, BATCH), jnp.float32),
    )(x)


def _sc_body(p1_hbm, p2_hbm, nodes_hbm, neight_hbm, out_hbm,
             nd_v, idx_v, acc_v, s0, s1, s2, s3, so):
    wid = lax.axis_index("s") * 2 + lax.axis_index("c")
    base = wid * BPW
    sems = [s0, s1, s2, s3]

    nd_cp = pltpu.async_copy(nodes_hbm.at[pl.ds(base, BPW)], nd_v, so)
    idx_cp = pltpu.async_copy(neight_hbm.at[:, pl.ds(base, BPW)], idx_v, so)
    nd_cp.wait()

    self_cps = [
        pltpu.async_copy(
            p1_hbm.at[nd_v.at[pl.ds(q * GRP, GRP)]],
            acc_v.at[pl.ds(q * GRP, GRP)], sems[q])
        for q in range(NGRP)
    ]
    idx_cp.wait()

    for q in range(NGRP):
        self_cps[q].wait()

        def add_round(j, carry, q=q):
            pltpu.async_copy(
                p2_hbm.at[idx_v.at[j, pl.ds(q * GRP, GRP)]],
                acc_v.at[pl.ds(q * GRP, GRP)], sems[q], add=True)
            return carry

        lax.fori_loop(0, NSAMP, add_round, 0)

    out_cps = []
    for q in range(NGRP):
        def drain_round(j, carry, q=q):
            pltpu.make_async_copy(
                p2_hbm.at[idx_v.at[0, pl.ds(q * GRP, GRP)]],
                acc_v.at[pl.ds(q * GRP, GRP)], sems[q]).wait()
            return carry

        lax.fori_loop(0, NSAMP, drain_round, 0)
        out_cps.append(pltpu.async_copy(
            acc_v.at[pl.ds(q * GRP, GRP)],
            out_hbm.at[pl.ds(base + q * GRP, GRP)], so))
    for cp in out_cps:
        cp.wait()


_sc_gather = functools.partial(
    pl.kernel,
    mesh=plsc.VectorSubcoreMesh(core_axis_name="c", subcore_axis_name="s"),
    out_type=jax.ShapeDtypeStruct((BATCH, EMBED), jnp.float32),
    scratch_types=[
        pltpu.VMEM((BPW,), jnp.int32),
        pltpu.VMEM((NSAMP, BPW), jnp.int32),
        pltpu.VMEM((BPW, EMBED), jnp.float32),
        pltpu.SemaphoreType.DMA,
        pltpu.SemaphoreType.DMA,
        pltpu.SemaphoreType.DMA,
        pltpu.SemaphoreType.DMA,
        pltpu.SemaphoreType.DMA,
    ],
)(_sc_body)


def kernel(nodes, neigh_idx, features, weight):
    w1t = weight[:, :D_FEAT].T
    w2t = weight[:, D_FEAT:].T * (1.0 / NSAMP)
    p1, p2 = _project(features, w1t, w2t)
    neight = neigh_idx.T.astype(jnp.int32)
    pre = _sc_gather(p1, p2, nodes.astype(jnp.int32), neight)
    return _relu_transpose(pre)


# speedup vs baseline: 1.1190x; 1.1176x over previous
import functools

import jax
import jax.numpy as jnp
from jax import lax
from jax.experimental import pallas as pl
from jax.experimental.pallas import tpu as pltpu
from jax.experimental.pallas import tpu_sc as plsc

N_NODES = 50000
D_FEAT = 256
EMBED = 128
BATCH = 16384
NSAMP = 32

NUM_WORKERS = 32
BPW = BATCH // NUM_WORKERS
GRP = 128
NGRP = BPW // GRP


def _proj_body(f_ref, w1_ref, w2_ref, p1_ref, p2_ref):
    f = f_ref[...]
    p1_ref[...] = jnp.dot(f, w1_ref[...], preferred_element_type=jnp.float32)
    p2_ref[...] = jnp.dot(f, w2_ref[...], preferred_element_type=jnp.float32)


def _project(features, w1t, w2t):
    rows = 2000
    return pl.pallas_call(
        _proj_body,
        grid=(N_NODES // rows,),
        in_specs=[
            pl.BlockSpec((rows, D_FEAT), lambda i: (i, 0)),
            pl.BlockSpec((D_FEAT, EMBED), lambda i: (0, 0)),
            pl.BlockSpec((D_FEAT, EMBED), lambda i: (0, 0)),
        ],
        out_specs=[
            pl.BlockSpec((rows, EMBED), lambda i: (i, 0)),
            pl.BlockSpec((rows, EMBED), lambda i: (i, 0)),
        ],
        out_shape=[jax.ShapeDtypeStruct((N_NODES, EMBED), jnp.float32)] * 2,
    )(features, w1t, w2t)


def _relu_t_body(x_ref, o_ref):
    o_ref[...] = jnp.maximum(x_ref[...].T, 0.0)


def _relu_transpose(x):
    cols = 4096
    return pl.pallas_call(
        _relu_t_body,
        grid=(BATCH // cols,),
        in_specs=[pl.BlockSpec((cols, EMBED), lambda i: (i, 0))],
        out_specs=pl.BlockSpec((EMBED, cols), lambda i: (0, i)),
        out_shape=jax.ShapeDtypeStruct((EMBED, BATCH), jnp.float32),
    )(x)


def _sc_body(p1_hbm, p2_hbm, nodes_hbm, neight_hbm, out_hbm,
             nd_v, idx_v, acc_v, sem):
    wid = lax.axis_index("s") * 2 + lax.axis_index("c")
    base = wid * BPW

    nd_cp = pltpu.async_copy(nodes_hbm.at[pl.ds(base, BPW)], nd_v, sem)
    idx_cp = pltpu.async_copy(neight_hbm.at[:, pl.ds(base, BPW)], idx_v, sem)

    zero = jnp.zeros((16,), jnp.float32)

    def zero_rows(r, carry):
        for u in range(8):
            for f in range(EMBED // 16):
                acc_v[r * 8 + u, pl.ds(f * 16, 16)] = zero
        return carry

    lax.fori_loop(0, BPW // 8, zero_rows, 0)
    nd_cp.wait()
    idx_cp.wait()

    for q in range(NGRP):
        pltpu.async_copy(
            p1_hbm.at[nd_v.at[pl.ds(q * GRP, GRP)]],
            acc_v.at[pl.ds(q * GRP, GRP)], sem, add=True)

    def add_round(j, carry):
        for q in range(NGRP):
            pltpu.async_copy(
                p2_hbm.at[idx_v.at[j, pl.ds(q * GRP, GRP)]],
                acc_v.at[pl.ds(q * GRP, GRP)], sem, add=True)
        return carry

    lax.fori_loop(0, NSAMP, add_round, 0)

    def drain_round(j, carry):
        for q in range(NGRP):
            pltpu.make_async_copy(
                p2_hbm.at[idx_v.at[0, pl.ds(q * GRP, GRP)]],
                acc_v.at[pl.ds(q * GRP, GRP)], sem).wait()
        return carry

    lax.fori_loop(0, NSAMP + 1, drain_round, 0)

    pltpu.sync_copy(acc_v, out_hbm.at[pl.ds(base, BPW)])


_sc_gather = functools.partial(
    pl.kernel,
    mesh=plsc.VectorSubcoreMesh(core_axis_name="c", subcore_axis_name="s"),
    out_type=jax.ShapeDtypeStruct((BATCH, EMBED), jnp.float32),
    scratch_types=[
        pltpu.VMEM((BPW,), jnp.int32),
        pltpu.VMEM((NSAMP, BPW), jnp.int32),
        pltpu.VMEM((BPW, EMBED), jnp.float32),
        pltpu.SemaphoreType.DMA,
    ],
)(_sc_body)


def kernel(nodes, neigh_idx, features, weight):
    w1t = weight[:, :D_FEAT].T
    w2t = weight[:, D_FEAT:].T * (1.0 / NSAMP)
    p1, p2 = _project(features, w1t, w2t)
    neight = neigh_idx.T.astype(jnp.int32)
    pre = _sc_gather(p1, p2, nodes.astype(jnp.int32), neight)
    return _relu_transpose(pre)


# speedup vs baseline: 1.1790x; 1.0536x over previous
import functools

import jax
import jax.numpy as jnp
from jax import lax
from jax.experimental import pallas as pl
from jax.experimental.pallas import tpu as pltpu
from jax.experimental.pallas import tpu_sc as plsc

N_NODES = 50000
D_FEAT = 256
EMBED = 128
BATCH = 16384
NSAMP = 32

NUM_WORKERS = 32
BPW = BATCH // NUM_WORKERS
GRP = 128
NGRP = BPW // GRP


def _proj_body(f_ref, w1_ref, w2_ref, p1_ref, p2_ref):
    f = f_ref[...]
    p1_ref[...] = jnp.dot(f, w1_ref[...], preferred_element_type=jnp.float32)
    p2_ref[...] = jnp.dot(f, w2_ref[...], preferred_element_type=jnp.float32)


def _project(features, w1t, w2t):
    rows = 5000
    return pl.pallas_call(
        _proj_body,
        grid=(N_NODES // rows,),
        in_specs=[
            pl.BlockSpec((rows, D_FEAT), lambda i: (i, 0)),
            pl.BlockSpec((D_FEAT, EMBED), lambda i: (0, 0)),
            pl.BlockSpec((D_FEAT, EMBED), lambda i: (0, 0)),
        ],
        out_specs=[
            pl.BlockSpec((rows, EMBED), lambda i: (i, 0)),
            pl.BlockSpec((rows, EMBED), lambda i: (i, 0)),
        ],
        out_shape=[jax.ShapeDtypeStruct((N_NODES, EMBED), jnp.float32)] * 2,
    )(features, w1t, w2t)


def _relu_t_body(x_ref, o_ref):
    o_ref[...] = jnp.maximum(x_ref[...].T, 0.0)


def _relu_transpose(x):
    cols = 8192
    return pl.pallas_call(
        _relu_t_body,
        grid=(BATCH // cols,),
        in_specs=[pl.BlockSpec((cols, EMBED), lambda i: (i, 0))],
        out_specs=pl.BlockSpec((EMBED, cols), lambda i: (0, i)),
        out_shape=jax.ShapeDtypeStruct((EMBED, BATCH), jnp.float32),
    )(x)


def _sc_body(p1_hbm, p2_hbm, nodes_hbm, neight_hbm, out_hbm,
             nd_v, idx_v, acc_v, sem):
    wid = lax.axis_index("s") * 2 + lax.axis_index("c")
    base = wid * BPW

    nd_cp = pltpu.async_copy(nodes_hbm.at[pl.ds(base, BPW)], nd_v, sem)
    idx_cp = pltpu.async_copy(neight_hbm.at[:, pl.ds(base, BPW)], idx_v, sem)

    zero = jnp.zeros((16,), jnp.float32)

    def zero_rows(r, carry):
        for u in range(8):
            for f in range(EMBED // 16):
                acc_v[r * 8 + u, pl.ds(f * 16, 16)] = zero
        return carry

    lax.fori_loop(0, BPW // 8, zero_rows, 0)
    nd_cp.wait()
    idx_cp.wait()

    for q in range(NGRP):
        pltpu.async_copy(
            p1_hbm.at[nd_v.at[pl.ds(q * GRP, GRP)]],
            acc_v.at[pl.ds(q * GRP, GRP)], sem, add=True)

    def add_round(j, carry):
        for q in range(NGRP):
            pltpu.async_copy(
                p2_hbm.at[idx_v.at[j, pl.ds(q * GRP, GRP)]],
                acc_v.at[pl.ds(q * GRP, GRP)], sem, add=True)
        return carry

    lax.fori_loop(0, NSAMP, add_round, 0)

    def drain_round(j, carry):
        for q in range(NGRP):
            pltpu.make_async_copy(
                p2_hbm.at[idx_v.at[0, pl.ds(q * GRP, GRP)]],
                acc_v.at[pl.ds(q * GRP, GRP)], sem).wait()
        return carry

    lax.fori_loop(0, NSAMP + 1, drain_round, 0)

    pltpu.sync_copy(acc_v, out_hbm.at[pl.ds(base, BPW)])


_sc_gather = functools.partial(
    pl.kernel,
    mesh=plsc.VectorSubcoreMesh(core_axis_name="c", subcore_axis_name="s"),
    out_type=jax.ShapeDtypeStruct((BATCH, EMBED), jnp.float32),
    scratch_types=[
        pltpu.VMEM((BPW,), jnp.int32),
        pltpu.VMEM((NSAMP, BPW), jnp.int32),
        pltpu.VMEM((BPW, EMBED), jnp.float32),
        pltpu.SemaphoreType.DMA,
    ],
)(_sc_body)


def kernel(nodes, neigh_idx, features, weight):
    w1t = weight[:, :D_FEAT].T
    w2t = weight[:, D_FEAT:].T * (1.0 / NSAMP)
    p1, p2 = _project(features, w1t, w2t)
    neight = neigh_idx.T.astype(jnp.int32)
    pre = _sc_gather(p1, p2, nodes.astype(jnp.int32), neight)
    return _relu_transpose(pre)
